# baseline (speedup 1.0000x reference)
import functools

import jax
import jax.numpy as jnp
from jax import lax
from jax.experimental import pallas as pl
from jax.experimental.pallas import tpu as pltpu
from jax.experimental.pallas import tpu_sc as plsc

EMB = 16
BW = 128
SS = 4
NC = 2
NS = 16
NW = NC * NS
PAD = 0


def _emb_body(seq, ids4d_hbm, table_hbm, out_hbm, idx_v, rows_v, tp_v,
              si_sem, sg0, sg1, ss0, ss1):
    nt = seq // SS
    nsb = seq // 8
    wid = lax.axis_index("s") * NC + lax.axis_index("c")
    for sb in range(nsb):
        pltpu.make_async_copy(
            ids4d_hbm.at[sb, wid], idx_v.at[sb], si_sem).start()
    for sb in range(nsb):
        pltpu.make_async_copy(
            ids4d_hbm.at[sb, wid], idx_v.at[sb], si_sem).wait()

    iota = lax.iota(jnp.int32, 16)
    zeros = jnp.zeros((16,), jnp.float32)
    sg = (sg0, sg1)
    ss = (ss0, ss1)

    def gather_q(t, p, q):
        s = t * SS + q
        return pltpu.make_async_copy(
            table_hbm.at[idx_v.at[s // 8, s % 8]],
            rows_v.at[pl.ds((p * SS + q) * BW, BW)], sg[p])

    def write_qe(t, p, q, eb):
        return pltpu.make_async_copy(
            tp_v.at[p, q, pl.ds(eb * 8, 8)],
            out_hbm.at[t * SS + q, eb, wid], ss[p])

    def fix(t, p):
        m = None
        for g in range(SS * BW // 16):
            s = t * SS + g // (BW // 16)
            iv = idx_v[s // 8, s % 8, pl.ds((g % (BW // 16)) * 16, 16)]
            m = iv if m is None else jnp.minimum(m, iv)
        cnt = plsc.all_reduce_population_count(m == PAD)

        @pl.when(cnt[0] > 0)
        def _():
            for q in range(SS):
                base = (p * SS + q) * BW
                for k in range(BW // 16):
                    s = t * SS + q
                    msk = idx_v[s // 8, s % 8, pl.ds(k * 16, 16)] == PAD
                    rows = iota + (base + k * 16)
                    for c in range(EMB):
                        plsc.store_scatter(
                            rows_v, [rows, jnp.full((16,), c, jnp.int32)],
                            zeros, mask=msk)

    def transpose_q(t, p, q):
        base = (p * SS + q) * BW
        for k in range(BW // 16):
            rowv = iota + (base + k * 16)
            vals = [
                plsc.load_gather(rows_v, [rowv, jnp.full((16,), e, jnp.int32)])
                for e in range(EMB)
            ]
            for e in range(EMB):
                tp_v[p, q, e, pl.ds(k * 16, 16)] = vals[e]

    for q in range(SS):
        gather_q(0, 0, q).start()

    def body(t2, carry):
        for p in range(2):
            t = t2 * 2 + p

            @pl.when(t < nt)
            def _():
                @pl.when(t + 1 < nt)
                def _():
                    @pl.when(t >= 1)
                    def _():
                        for q in range(SS):
                            for eb in range(2):
                                write_qe(t - 1, 1 - p, q, eb).wait()
                    for q in range(SS):
                        gather_q(t + 1, 1 - p, q).start()

                for q in range(SS):
                    gather_q(t, p, q).wait()
                fix(t, p)
                for q in range(SS):
                    transpose_q(t, p, q)
                for q in range(SS):
                    for eb in range(2):
                        write_qe(t, p, q, eb).start()
        return carry

    lax.fori_loop(0, (nt + 2) // 2, body, 0)
    for q in range(SS):
        for eb in range(2):
            write_qe(nt - 2, (nt - 2) % 2, q, eb).wait()
            write_qe(nt - 1, (nt - 1) % 2, q, eb).wait()


def kernel(input, weight):
    ids = input.astype(jnp.int32)
    b, seq = ids.shape
    nb = b // BW
    ids4d = ids.T.reshape(seq // 8, 8, nb, BW).transpose(0, 2, 1, 3)

    mesh = plsc.VectorSubcoreMesh(core_axis_name="c", subcore_axis_name="s")
    run = pl.kernel(
        functools.partial(_emb_body, seq),
        mesh=mesh,
        compiler_params=pltpu.CompilerParams(
            use_tc_tiling_on_sc=False, needs_layout_passes=False),
        out_type=jax.ShapeDtypeStruct((seq, EMB // 8, nb, 8, BW),
                                      jnp.float32),
        scratch_types=[
            pltpu.VMEM((seq // 8, 8, BW), jnp.int32),
            pltpu.VMEM((2 * SS * BW, EMB), jnp.float32),
            pltpu.VMEM((2, SS, EMB, BW), jnp.float32),
            pltpu.SemaphoreType.DMA,
            pltpu.SemaphoreType.DMA,
            pltpu.SemaphoreType.DMA,
            pltpu.SemaphoreType.DMA,
            pltpu.SemaphoreType.DMA,
        ],
    )
    out_phys = run(ids4d, weight)
    return out_phys.transpose(2, 4, 0, 1, 3).reshape(b, seq, EMB)
